# Initial kernel scaffold; baseline (speedup 1.0000x reference)
import jax
import jax.numpy as jnp
from jax.experimental import pallas as pl


def kernel(event_times, cu_seqlens, node_pairs, x0, v, beta):
    raise NotImplementedError("write your pallas kernel here")


# speedup vs baseline: 1.3447x; 1.3447x over previous
import functools
import jax
import jax.numpy as jnp
from jax import lax
from jax.experimental import pallas as pl
from jax.experimental.pallas import tpu as pltpu
from jax.experimental.pallas import tpu_sc as plsc

NC, NS, L = 2, 16, 16
NW = NC * NS
PW_ = 1.0

f32 = jnp.float32
i32 = jnp.int32


def _build_sc_call(T, P, BINS, NN, EV_CHUNK=1024):
    PB = P // NW
    NBK = BINS * PB
    PBF = 2 * PB
    NG = PBF // L
    NROW = NN // 8
    bwf = f32(1.0 / BINS)
    half = f32(0.5 / BINS)
    mesh = plsc.VectorSubcoreMesh(core_axis_name="c", subcore_axis_name="s")

    @functools.partial(
        pl.kernel, mesh=mesh,
        out_type=jax.ShapeDtypeStruct((NW, L), f32),
        compiler_params=pltpu.CompilerParams(
            needs_layout_passes=False, use_tc_tiling_on_sc=False),
        scratch_types=[
            pltpu.VMEM((136,), i32),
            pltpu.VMEM((PB,), i32),
            pltpu.VMEM((PB,), i32),
            pltpu.VMEM((PB,), i32),
            pltpu.VMEM((PB,), i32),
            pltpu.VMEM((PB,), i32),
            pltpu.VMEM((PB,), i32),
            pltpu.VMEM((PBF,), i32),
            pltpu.VMEM((PBF,), i32),
            pltpu.VMEM((PB, 16), f32),
            pltpu.VMEM((PB, 16), f32),
            pltpu.VMEM((PBF,), f32),
            pltpu.VMEM((PB, 16), f32),
            pltpu.VMEM((PB, 16), f32),
            pltpu.VMEM((NBK,), f32),
            pltpu.VMEM((NBK,), f32),
            pltpu.VMEM((NBK,), f32),
            pltpu.VMEM((EV_CHUNK,), f32),
            pltpu.VMEM((L,), f32),
            pltpu.VMEM((L,), f32),
            pltpu.SemaphoreType.DMA,
            pltpu.SemaphoreType.DMA,
        ],
    )
    def sc_call(et_hbm, cs_hbm, np_hbm, x016_hbm, v16_hbm, out_hbm,
                cs_loc, mi_v, mj_v, rbi_v, rbj_v, ixi_v, ixj_v,
                coli_v, colj_v, x0i_v, x0j_v, dx0_v, vi_v, vj_v,
                cnt_v, s1_v, s2_v, ev_v, shuf_v, orow_v, semA, semB):
        wid = lax.axis_index("s") * NC + lax.axis_index("c")
        pbase = wid * PB
        LANE = lax.iota(i32, L)
        DUPV = lax.shift_right_logical(LANE, 1)
        SWAPV = lax.bitwise_xor(LANE, 1)
        COLV = lax.bitwise_and(LANE, 1)

        pltpu.sync_copy(cs_hbm.at[pl.ds(pbase, 136)], cs_loc)
        pltpu.sync_copy(np_hbm.at[0, pl.ds(pbase, PB)], mi_v)
        pltpu.sync_copy(np_hbm.at[1, pl.ds(pbase, PB)], mj_v)

        for g in range(PB // L):
            sl = pl.ds(g * L, L)
            rbi_v[sl] = lax.shift_right_logical(mi_v[sl], 3)
            rbj_v[sl] = lax.shift_right_logical(mj_v[sl], 3)
        for g in range(NG):
            sl = pl.ds(g * L, L)
            ni = plsc.load_gather(mi_v, [g * 8 + DUPV])
            nj = plsc.load_gather(mj_v, [g * 8 + DUPV])
            coli_v[sl] = 2 * lax.bitwise_and(ni, 7) + COLV
            colj_v[sl] = 2 * lax.bitwise_and(nj, 7) + COLV

        pltpu.async_copy(x016_hbm.at[rbi_v], x0i_v, semA).wait()
        pltpu.async_copy(x016_hbm.at[rbj_v], x0j_v, semB).wait()

        e0 = cs_loc[pl.ds(0, L)][0]
        e1 = cs_loc[pl.ds(120, L)][8]

        def zero_body(k, _):
            z = jnp.zeros((L,), f32)
            cnt_v[pl.ds(k * L, L)] = z
            s1_v[pl.ds(k * L, L)] = z
            s2_v[pl.ds(k * L, L)] = z
            return 0
        lax.fori_loop(0, NBK // L, zero_body, 0)

        pr0 = jnp.zeros((L,), f32)
        for g in range(NG):
            sl = pl.ds(g * L, L)
            rg = g * 8 + DUPV
            xi = plsc.load_gather(x0i_v, [rg, coli_v[sl]])
            xj = plsc.load_gather(x0j_v, [rg, colj_v[sl]])
            dx0_v[sl] = xi - xj
            pr0 = pr0 + xi * xi + xj * xj

        e0_al = (e0 // 8) * 8
        nch = (e1 - e0_al + (EV_CHUNK - 1)) // EV_CHUNK
        ones = jnp.ones((L,), f32)

        def ev_chunk(ch, _):
            chb = e0_al + ch * EV_CHUNK
            pltpu.async_copy(et_hbm.at[pl.ds(chb, EV_CHUNK)], ev_v, semA).wait()
            for g in range(EV_CHUNK // L):
                gidx = chb + g * L + LANE
                valid = jnp.logical_and(gidx >= e0, gidx < e1)
                t_e = ev_v[pl.ds(g * L, L)]
                pos = jnp.zeros((L,), i32)
                for w in (64, 32, 16, 8, 4, 2, 1):
                    cand = pos + w
                    cv = plsc.load_gather(cs_loc, [cand])
                    pos = jnp.where(cv <= gidx, cand, pos)
                b = jnp.clip((t_e / bwf).astype(i32), 0, BINS - 1)
                r = t_e - b.astype(f32) * bwf
                bucket = b * PB + pos
                plsc.addupdate_scatter(cnt_v, [bucket], ones, mask=valid)
                plsc.addupdate_scatter(s1_v, [bucket], r, mask=valid)
                plsc.addupdate_scatter(s2_v, [bucket], r * r, mask=valid)
            return 0
        lax.fori_loop(0, nch, ev_chunk, 0)

        def bin_body(b, carry):
            cums = carry[0]
            ev_acc, int_acc, pr_acc = carry[1], carry[2], carry[3]
            boff = b * NROW
            for g in range(PB // L):
                sl = pl.ds(g * L, L)
                ixi_v[sl] = rbi_v[sl] + boff
                ixj_v[sl] = rbj_v[sl] + boff
            pltpu.async_copy(v16_hbm.at[ixi_v], vi_v, semA).wait()
            pltpu.async_copy(v16_hbm.at[ixj_v], vj_v, semB).wait()
            new_cums = []
            for g in range(NG):
                sl = pl.ds(g * L, L)
                rg = g * 8 + DUPV
                vi = plsc.load_gather(vi_v, [rg, coli_v[sl]])
                vj = plsc.load_gather(vj_v, [rg, colj_v[sl]])
                dv = vi - vj
                e_ = dx0_v[sl] + bwf * cums[g]
                new_cums.append(cums[g] + dv)
                pa = e_ * e_
                pb2 = e_ * dv
                pc = dv * dv
                dup = b * PB + g * 8 + DUPV
                c_ = plsc.load_gather(cnt_v, [dup])
                s1 = plsc.load_gather(s1_v, [dup])
                s2 = plsc.load_gather(s2_v, [dup])
                ev_acc = ev_acc + pa * c_ + 2.0 * pb2 * s1 + pc * s2
                d2l = pa + pb2 * bwf + pc * (half * half)
                shuf_v[...] = d2l
                d2s = plsc.load_gather(shuf_v, [SWAPV])
                int_acc = int_acc + jnp.exp(-(d2l + d2s))
                pr_acc = pr_acc + vi * vi + vj * vj
            return (tuple(new_cums), ev_acc, int_acc, pr_acc)

        z = jnp.zeros((L,), f32)
        init = (tuple(z for _ in range(NG)), z, z, pr0)
        _, ev_acc, int_acc, pr_acc = lax.fori_loop(0, BINS, bin_body, init)

        ev_s = jnp.sum(ev_acc)
        int_s = jnp.sum(int_acc)
        pr_s = jnp.sum(pr_acc)
        orow_v[...] = (jnp.where(LANE == 0, ev_s, f32(0.0))
                       + jnp.where(LANE == 1, int_s, f32(0.0))
                       + jnp.where(LANE == 2, pr_s, f32(0.0)))
        pltpu.sync_copy(orow_v, out_hbm.at[wid])

    return sc_call


def kernel(event_times, cu_seqlens, node_pairs, x0, v, beta):
    T = event_times.shape[0]
    P = node_pairs.shape[1]
    BINS, NN, D = v.shape
    EV_CHUNK = 1024
    bw = 1.0 / BINS

    et_pad = jnp.concatenate(
        [event_times, jnp.zeros((EV_CHUNK + 8,), f32)])
    cs_pad = jnp.concatenate(
        [cu_seqlens.astype(i32), jnp.full((7,), T, i32)])
    v16 = v.reshape(BINS * NN * D // 16, 16)
    x016 = x0.reshape(NN * D // 16, 16)

    sc_call = _build_sc_call(T, P, BINS, NN, EV_CHUNK)
    parts = sc_call(et_pad, cs_pad, node_pairs, x016, v16)

    ev_delta2 = jnp.sum(parts[:, 0])
    int_raw = jnp.sum(parts[:, 1]) * 0.5
    prior_raw = jnp.sum(parts[:, 2])

    b0 = beta[0]
    integral_term = jnp.exp(b0) * int_raw * bw
    events_term = T * b0 - ev_delta2
    prior_term = 0.5 * PW_ * prior_raw
    return integral_term - events_term + prior_term
